# baseline (speedup 1.0000x reference)
import jax
import jax.numpy as jnp
from jax.experimental import pallas as pl
from jax.experimental.pallas import tpu as pltpu

N = 1000
K = 32
RB = 200
NBLK = N // RB
P = 1001
MLPH = 512
HID = 128
OUT = 128


def _cumsum_lanes(x):
    rows, n = x.shape
    s = 1
    while s < n:
        shifted = jnp.concatenate(
            [jnp.zeros((rows, s), x.dtype), x[:, : n - s]], axis=1
        )
        x = x + shifted
        s *= 2
    return x


def _count_ge(key, c, ones8):
    rows, n = key.shape
    m = jnp.where(key >= c, 1.0, 0.0)
    t = m[:, 0:128]
    for j in range(1, 7):
        t = t + m[:, j * 128:(j + 1) * 128]
    tail = jnp.concatenate(
        [m[:, 896:n], jnp.zeros((rows, 1024 - n), jnp.float32)], axis=1
    )
    t = t + tail
    return jnp.dot(t, ones8, preferred_element_type=jnp.float32)[:, 0:1]


def _topk_search(adj):
    rows, n = adj.shape
    b = jax.lax.bitcast_convert_type(adj, jnp.int32)
    key = jnp.where(b >= 0, b, b ^ jnp.int32(0x7FFFFFFF))
    ones8 = jnp.ones((128, 8), jnp.float32)
    kf = jnp.float32(K)

    def body(i, carry):
        p, cnt = carry
        c = p + (jnp.int32(1) << (jnp.int32(31) - i))
        cc = _count_ge(key, c, ones8)
        ok = cc >= kf
        return jnp.where(ok, c, p), jnp.where(ok, cc, cnt)

    p0 = jnp.full((rows, 1), jnp.int32(-2147483648))
    cnt0 = jnp.full((rows, 1), jnp.float32(n))
    p, cnt = jax.lax.fori_loop(0, 32, body, (p0, cnt0))
    return key, p, cnt, kf


def _body(q_ref, p_ref, bb_ref, w1_ref, w1c_ref, b1_ref, w2_ref, b2_ref,
          ne_ref, wc1_ref, bc1_ref, wc2_ref, bc2_ref, gamma_ref, beta_ref,
          out_ref, a_scr):
    i = pl.program_id(0)

    h = jnp.dot(q_ref[...], w1_ref[0:N, :], preferred_element_type=jnp.float32)
    h = h + jnp.dot(p_ref[...], w1_ref[N:N + P, :],
                    preferred_element_type=jnp.float32)
    h = h + jnp.dot(bb_ref[...], w1c_ref[...],
                    preferred_element_type=jnp.float32)
    h = jnp.maximum(h + b1_ref[...], 0.0)
    adj = jnp.dot(h, w2_ref[...], preferred_element_type=jnp.float32) + b2_ref[...]

    key, p, cnt, kf = _topk_search(adj)
    rowsl = pl.ds(i * RB, RB)
    a_scr[rowsl, :] = jnp.where(key >= p, adj, 0.0)

    any_tie = jnp.sum(jnp.where(cnt != kf, 1.0, 0.0)) > 0.0

    @pl.when(any_tie)
    def _exact_ties():
        gt = key > p
        eq = key == p
        cgt = jnp.sum(gt.astype(jnp.int32), axis=1, keepdims=True)
        need = K - cgt
        eqcs = _cumsum_lanes(eq.astype(jnp.int32))
        mask = gt | (eq & ((cnt == kf) | (eqcs <= need)))
        a_scr[rowsl, :] = jnp.where(mask, adj, 0.0)

    @pl.when(i == NBLK - 1)
    def _gcn():
        A = a_scr[...]
        deg = 1.0 + jnp.sum(A, axis=1, keepdims=True)
        dinv = jnp.where(deg > 0, jax.lax.rsqrt(deg), 0.0)

        h1 = jnp.dot(ne_ref[...], wc1_ref[...],
                     preferred_element_type=jnp.float32)
        agg1 = (
            dinv * jnp.dot(A, dinv * h1, preferred_element_type=jnp.float32)
            + (dinv * dinv) * h1
            + bc1_ref[...]
        )

        mean = jnp.sum(agg1, axis=0, keepdims=True) / N
        var = jnp.sum((agg1 - mean) ** 2, axis=0, keepdims=True) / N
        o1 = (gamma_ref[...] * (agg1 - mean) * jax.lax.rsqrt(var + 1e-5)
              + beta_ref[...])
        o1 = jnp.maximum(o1, 0.0)

        h2 = jnp.dot(o1, wc2_ref[...], preferred_element_type=jnp.float32)
        out_ref[...] = (
            dinv * jnp.dot(A, dinv * h2, preferred_element_type=jnp.float32)
            + (dinv * dinv) * h2
            + bc2_ref[...]
        )


def kernel(probs, bbox_coords, query_embeddings, node_embeddings,
           W1, b1, W2, b2, Wc1, bc1, Wc2, bc2, gamma, beta):
    f32 = jnp.float32
    W1c = W1[N + P:, :]

    const = lambda i: (0, 0)
    out = pl.pallas_call(
        _body,
        grid=(NBLK,),
        in_specs=[
            pl.BlockSpec((RB, N), lambda i: (i, 0)),
            pl.BlockSpec((RB, P), lambda i: (i, 0)),
            pl.BlockSpec((RB, 4), lambda i: (i, 0)),
            pl.BlockSpec((N + P + 4, MLPH), const),
            pl.BlockSpec((4, MLPH), const),
            pl.BlockSpec((1, MLPH), const),
            pl.BlockSpec((MLPH, N), const),
            pl.BlockSpec((1, N), const),
            pl.BlockSpec((N, N), const),
            pl.BlockSpec((N, HID), const),
            pl.BlockSpec((1, HID), const),
            pl.BlockSpec((HID, OUT), const),
            pl.BlockSpec((1, OUT), const),
            pl.BlockSpec((1, HID), const),
            pl.BlockSpec((1, HID), const),
        ],
        out_specs=pl.BlockSpec((N, OUT), const),
        out_shape=jax.ShapeDtypeStruct((N, OUT), f32),
        scratch_shapes=[pltpu.VMEM((N, N), f32)],
        compiler_params=pltpu.CompilerParams(
            dimension_semantics=("arbitrary",),
        ),
    )(query_embeddings, probs, bbox_coords, W1, W1c,
      b1.reshape(1, MLPH), W2, b2.reshape(1, N),
      node_embeddings, Wc1, bc1.reshape(1, HID), Wc2, bc2.reshape(1, OUT),
      gamma.reshape(1, HID), beta.reshape(1, HID))
    return out


# speedup vs baseline: 1.3168x; 1.3168x over previous
import jax
import jax.numpy as jnp
from jax.experimental import pallas as pl
from jax.experimental.pallas import tpu as pltpu

N = 1000
K = 32
RB = 200
NBLK = N // RB
P = 1001
MLPH = 512
HID = 128
OUT = 128


def _cumsum_lanes(x):
    rows, n = x.shape
    s = 1
    while s < n:
        shifted = jnp.concatenate(
            [jnp.zeros((rows, s), x.dtype), x[:, : n - s]], axis=1
        )
        x = x + shifted
        s *= 2
    return x


def _count_ge(key, c, ones8):
    rows, n = key.shape
    m = jnp.where(key >= c, 1.0, 0.0)
    t = m[:, 0:128]
    for j in range(1, 7):
        t = t + m[:, j * 128:(j + 1) * 128]
    tail = jnp.concatenate(
        [m[:, 896:n], jnp.zeros((rows, 1024 - n), jnp.float32)], axis=1
    )
    t = t + tail
    return jnp.dot(t, ones8, preferred_element_type=jnp.float32)[:, 0:1]


def _topk_search(adj):
    rows, n = adj.shape
    b = jax.lax.bitcast_convert_type(adj, jnp.int32)
    key = jnp.where(b >= 0, b, b ^ jnp.int32(0x7FFFFFFF))
    ones8 = jnp.ones((128, 8), jnp.float32)
    kf = jnp.float32(K)

    def body(i, carry):
        p, cnt = carry
        c = p + (jnp.int32(1) << (jnp.int32(31) - i))
        cc = jnp.sum((key >= c).astype(jnp.float32), axis=1, keepdims=True)
        ok = cc >= kf
        return jnp.where(ok, c, p), jnp.where(ok, cc, cnt)

    p0 = jnp.full((rows, 1), jnp.int32(-2147483648))
    cnt0 = jnp.full((rows, 1), jnp.float32(n))
    p, cnt = jax.lax.fori_loop(0, 32, body, (p0, cnt0))
    return key, p, cnt, kf


def _body(q_ref, p_ref, bb_ref, w1_ref, w1c_ref, b1_ref, w2_ref, b2_ref,
          ne_ref, wc1_ref, bc1_ref, wc2_ref, bc2_ref, gamma_ref, beta_ref,
          out_ref, a_scr):
    i = pl.program_id(0)

    h = jnp.dot(q_ref[...], w1_ref[0:N, :], preferred_element_type=jnp.float32)
    h = h + jnp.dot(p_ref[...], w1_ref[N:N + P, :],
                    preferred_element_type=jnp.float32)
    h = h + jnp.dot(bb_ref[...], w1c_ref[...],
                    preferred_element_type=jnp.float32)
    h = jnp.maximum(h + b1_ref[...], 0.0)
    adj = jnp.dot(h, w2_ref[...], preferred_element_type=jnp.float32) + b2_ref[...]

    key, p, cnt, kf = _topk_search(adj)
    rowsl = pl.ds(i * RB, RB)
    a_scr[rowsl, :] = jnp.where(key >= p, adj, 0.0)

    any_tie = jnp.sum(jnp.where(cnt != kf, 1.0, 0.0)) > 0.0

    @pl.when(any_tie)
    def _exact_ties():
        gt = key > p
        eq = key == p
        cgt = jnp.sum(gt.astype(jnp.int32), axis=1, keepdims=True)
        need = K - cgt
        eqcs = _cumsum_lanes(eq.astype(jnp.int32))
        mask = gt | (eq & ((cnt == kf) | (eqcs <= need)))
        a_scr[rowsl, :] = jnp.where(mask, adj, 0.0)

    @pl.when(i == NBLK - 1)
    def _gcn():
        A = a_scr[...]
        deg = 1.0 + jnp.sum(A, axis=1, keepdims=True)
        dinv = jnp.where(deg > 0, jax.lax.rsqrt(deg), 0.0)

        h1 = jnp.dot(ne_ref[...], wc1_ref[...],
                     preferred_element_type=jnp.float32)
        agg1 = (
            dinv * jnp.dot(A, dinv * h1, preferred_element_type=jnp.float32)
            + (dinv * dinv) * h1
            + bc1_ref[...]
        )

        mean = jnp.sum(agg1, axis=0, keepdims=True) / N
        var = jnp.sum((agg1 - mean) ** 2, axis=0, keepdims=True) / N
        o1 = (gamma_ref[...] * (agg1 - mean) * jax.lax.rsqrt(var + 1e-5)
              + beta_ref[...])
        o1 = jnp.maximum(o1, 0.0)

        h2 = jnp.dot(o1, wc2_ref[...], preferred_element_type=jnp.float32)
        out_ref[...] = (
            dinv * jnp.dot(A, dinv * h2, preferred_element_type=jnp.float32)
            + (dinv * dinv) * h2
            + bc2_ref[...]
        )


def kernel(probs, bbox_coords, query_embeddings, node_embeddings,
           W1, b1, W2, b2, Wc1, bc1, Wc2, bc2, gamma, beta):
    f32 = jnp.float32
    W1c = W1[N + P:, :]

    const = lambda i: (0, 0)
    out = pl.pallas_call(
        _body,
        grid=(NBLK,),
        in_specs=[
            pl.BlockSpec((RB, N), lambda i: (i, 0)),
            pl.BlockSpec((RB, P), lambda i: (i, 0)),
            pl.BlockSpec((RB, 4), lambda i: (i, 0)),
            pl.BlockSpec((N + P + 4, MLPH), const),
            pl.BlockSpec((4, MLPH), const),
            pl.BlockSpec((1, MLPH), const),
            pl.BlockSpec((MLPH, N), const),
            pl.BlockSpec((1, N), const),
            pl.BlockSpec((N, N), const),
            pl.BlockSpec((N, HID), const),
            pl.BlockSpec((1, HID), const),
            pl.BlockSpec((HID, OUT), const),
            pl.BlockSpec((1, OUT), const),
            pl.BlockSpec((1, HID), const),
            pl.BlockSpec((1, HID), const),
        ],
        out_specs=pl.BlockSpec((N, OUT), const),
        out_shape=jax.ShapeDtypeStruct((N, OUT), f32),
        scratch_shapes=[pltpu.VMEM((N, N), f32)],
        compiler_params=pltpu.CompilerParams(
            dimension_semantics=("arbitrary",),
        ),
    )(query_embeddings, probs, bbox_coords, W1, W1c,
      b1.reshape(1, MLPH), W2, b2.reshape(1, N),
      node_embeddings, Wc1, bc1.reshape(1, HID), Wc2, bc2.reshape(1, OUT),
      gamma.reshape(1, HID), beta.reshape(1, HID))
    return out


# speedup vs baseline: 1.6043x; 1.2184x over previous
import jax
import jax.numpy as jnp
from jax.experimental import pallas as pl
from jax.experimental.pallas import tpu as pltpu

N = 1000
K = 32
RB = 200
NBLK = N // RB
P = 1001
MLPH = 512
HID = 128
OUT = 128


def _cumsum_lanes(x):
    rows, n = x.shape
    s = 1
    while s < n:
        shifted = jnp.concatenate(
            [jnp.zeros((rows, s), x.dtype), x[:, : n - s]], axis=1
        )
        x = x + shifted
        s *= 2
    return x


def _count_ge(key, c, ones8):
    rows, n = key.shape
    m = jnp.where(key >= c, 1.0, 0.0)
    t = m[:, 0:128]
    for j in range(1, 7):
        t = t + m[:, j * 128:(j + 1) * 128]
    tail = jnp.concatenate(
        [m[:, 896:n], jnp.zeros((rows, 1024 - n), jnp.float32)], axis=1
    )
    t = t + tail
    return jnp.dot(t, ones8, preferred_element_type=jnp.float32)[:, 0:1]


def _topk_search(adj):
    rows, n = adj.shape
    b = jax.lax.bitcast_convert_type(adj, jnp.int32)
    key = jnp.where(b >= 0, b, b ^ jnp.int32(0x7FFFFFFF))
    ones8 = jnp.ones((128, 8), jnp.float32)
    kf = jnp.float32(K)

    def body(i, carry):
        p, cnt = carry
        c = p + (jnp.int32(1) << (jnp.int32(31) - i))
        cc = jnp.sum((key >= c).astype(jnp.float32), axis=1, keepdims=True)
        ok = cc >= kf
        return jnp.where(ok, c, p), jnp.where(ok, cc, cnt)

    p0 = jnp.full((rows, 1), jnp.int32(-2147483648))
    cnt0 = jnp.full((rows, 1), jnp.float32(n))
    p, cnt = jax.lax.fori_loop(0, 32, body, (p0, cnt0), unroll=4)
    return key, p, cnt, kf


def _body(q_ref, p_ref, bb_ref, w1_ref, w1c_ref, b1_ref, w2_ref, b2_ref,
          ne_ref, wc1_ref, bc1_ref, wc2_ref, bc2_ref, gamma_ref, beta_ref,
          out_ref, a_scr):
    i = pl.program_id(0)

    h = jnp.dot(q_ref[...], w1_ref[0:N, :], preferred_element_type=jnp.float32)
    h = h + jnp.dot(p_ref[...], w1_ref[N:N + P, :],
                    preferred_element_type=jnp.float32)
    h = h + jnp.dot(bb_ref[...], w1c_ref[...],
                    preferred_element_type=jnp.float32)
    h = jnp.maximum(h + b1_ref[...], 0.0)
    adj = jnp.dot(h, w2_ref[...], preferred_element_type=jnp.float32) + b2_ref[...]

    key, p, cnt, kf = _topk_search(adj)
    rowsl = pl.ds(i * RB, RB)
    a_scr[rowsl, :] = jnp.where(key >= p, adj, 0.0)

    any_tie = jnp.sum(jnp.where(cnt != kf, 1.0, 0.0)) > 0.0

    @pl.when(any_tie)
    def _exact_ties():
        gt = key > p
        eq = key == p
        cgt = jnp.sum(gt.astype(jnp.int32), axis=1, keepdims=True)
        need = K - cgt
        eqcs = _cumsum_lanes(eq.astype(jnp.int32))
        mask = gt | (eq & ((cnt == kf) | (eqcs <= need)))
        a_scr[rowsl, :] = jnp.where(mask, adj, 0.0)

    @pl.when(i == NBLK - 1)
    def _gcn():
        A = a_scr[...]
        deg = 1.0 + jnp.sum(A, axis=1, keepdims=True)
        dinv = jnp.where(deg > 0, jax.lax.rsqrt(deg), 0.0)

        h1 = jnp.dot(ne_ref[...], wc1_ref[...],
                     preferred_element_type=jnp.float32)
        agg1 = (
            dinv * jnp.dot(A, dinv * h1, preferred_element_type=jnp.float32)
            + (dinv * dinv) * h1
            + bc1_ref[...]
        )

        mean = jnp.sum(agg1, axis=0, keepdims=True) / N
        var = jnp.sum((agg1 - mean) ** 2, axis=0, keepdims=True) / N
        o1 = (gamma_ref[...] * (agg1 - mean) * jax.lax.rsqrt(var + 1e-5)
              + beta_ref[...])
        o1 = jnp.maximum(o1, 0.0)

        h2 = jnp.dot(o1, wc2_ref[...], preferred_element_type=jnp.float32)
        out_ref[...] = (
            dinv * jnp.dot(A, dinv * h2, preferred_element_type=jnp.float32)
            + (dinv * dinv) * h2
            + bc2_ref[...]
        )


def kernel(probs, bbox_coords, query_embeddings, node_embeddings,
           W1, b1, W2, b2, Wc1, bc1, Wc2, bc2, gamma, beta):
    f32 = jnp.float32
    W1c = W1[N + P:, :]

    const = lambda i: (0, 0)
    out = pl.pallas_call(
        _body,
        grid=(NBLK,),
        in_specs=[
            pl.BlockSpec((RB, N), lambda i: (i, 0)),
            pl.BlockSpec((RB, P), lambda i: (i, 0)),
            pl.BlockSpec((RB, 4), lambda i: (i, 0)),
            pl.BlockSpec((N + P + 4, MLPH), const),
            pl.BlockSpec((4, MLPH), const),
            pl.BlockSpec((1, MLPH), const),
            pl.BlockSpec((MLPH, N), const),
            pl.BlockSpec((1, N), const),
            pl.BlockSpec((N, N), const),
            pl.BlockSpec((N, HID), const),
            pl.BlockSpec((1, HID), const),
            pl.BlockSpec((HID, OUT), const),
            pl.BlockSpec((1, OUT), const),
            pl.BlockSpec((1, HID), const),
            pl.BlockSpec((1, HID), const),
        ],
        out_specs=pl.BlockSpec((N, OUT), const),
        out_shape=jax.ShapeDtypeStruct((N, OUT), f32),
        scratch_shapes=[pltpu.VMEM((N, N), f32)],
        compiler_params=pltpu.CompilerParams(
            dimension_semantics=("arbitrary",),
        ),
    )(query_embeddings, probs, bbox_coords, W1, W1c,
      b1.reshape(1, MLPH), W2, b2.reshape(1, N),
      node_embeddings, Wc1, bc1.reshape(1, HID), Wc2, bc2.reshape(1, OUT),
      gamma.reshape(1, HID), beta.reshape(1, HID))
    return out


# speedup vs baseline: 1.6726x; 1.0425x over previous
import jax
import jax.numpy as jnp
from jax.experimental import pallas as pl
from jax.experimental.pallas import tpu as pltpu

N = 1000
K = 32
RB = 200
NBLK = N // RB
P = 1001
MLPH = 512
HID = 128
OUT = 128


def _cumsum_lanes(x):
    rows, n = x.shape
    s = 1
    while s < n:
        shifted = jnp.concatenate(
            [jnp.zeros((rows, s), x.dtype), x[:, : n - s]], axis=1
        )
        x = x + shifted
        s *= 2
    return x


def _count_ge(key, c, ones8):
    rows, n = key.shape
    m = jnp.where(key >= c, 1.0, 0.0)
    t = m[:, 0:128]
    for j in range(1, 7):
        t = t + m[:, j * 128:(j + 1) * 128]
    tail = jnp.concatenate(
        [m[:, 896:n], jnp.zeros((rows, 1024 - n), jnp.float32)], axis=1
    )
    t = t + tail
    return jnp.dot(t, ones8, preferred_element_type=jnp.float32)[:, 0:1]


def _topk_search(adj):
    rows, n = adj.shape
    b = jax.lax.bitcast_convert_type(adj, jnp.int32)
    key = jnp.where(b >= 0, b, b ^ jnp.int32(0x7FFFFFFF))
    ones8 = jnp.ones((128, 8), jnp.float32)
    kf = jnp.float32(K)

    def body(i, carry):
        p, cnt = carry
        c = p + (jnp.int32(1) << (jnp.int32(31) - i))
        cc = jnp.sum((key >= c).astype(jnp.float32), axis=1, keepdims=True)
        ok = cc >= kf
        return jnp.where(ok, c, p), jnp.where(ok, cc, cnt)

    p0 = jnp.full((rows, 1), jnp.int32(-2147483648))
    cnt0 = jnp.full((rows, 1), jnp.float32(n))
    p, cnt = jax.lax.fori_loop(0, 32, body, (p0, cnt0), unroll=8)
    return key, p, cnt, kf


def _body(q_ref, p_ref, bb_ref, w1_ref, w1c_ref, b1_ref, w2_ref, b2_ref,
          ne_ref, wc1_ref, bc1_ref, wc2_ref, bc2_ref, gamma_ref, beta_ref,
          out_ref, a_scr):
    i = pl.program_id(0)

    h = jnp.dot(q_ref[...], w1_ref[0:N, :], preferred_element_type=jnp.float32)
    h = h + jnp.dot(p_ref[...], w1_ref[N:N + P, :],
                    preferred_element_type=jnp.float32)
    h = h + jnp.dot(bb_ref[...], w1c_ref[...],
                    preferred_element_type=jnp.float32)
    h = jnp.maximum(h + b1_ref[...], 0.0)
    adj = jnp.dot(h, w2_ref[...], preferred_element_type=jnp.float32) + b2_ref[...]

    key, p, cnt, kf = _topk_search(adj)
    rowsl = pl.ds(i * RB, RB)
    a_scr[rowsl, :] = jnp.where(key >= p, adj, 0.0)

    any_tie = jnp.sum(jnp.where(cnt != kf, 1.0, 0.0)) > 0.0

    @pl.when(any_tie)
    def _exact_ties():
        gt = key > p
        eq = key == p
        cgt = jnp.sum(gt.astype(jnp.int32), axis=1, keepdims=True)
        need = K - cgt
        eqcs = _cumsum_lanes(eq.astype(jnp.int32))
        mask = gt | (eq & ((cnt == kf) | (eqcs <= need)))
        a_scr[rowsl, :] = jnp.where(mask, adj, 0.0)

    @pl.when(i == NBLK - 1)
    def _gcn():
        A = a_scr[...]
        deg = 1.0 + jnp.sum(A, axis=1, keepdims=True)
        dinv = jnp.where(deg > 0, jax.lax.rsqrt(deg), 0.0)

        h1 = jnp.dot(ne_ref[...], wc1_ref[...],
                     preferred_element_type=jnp.float32)
        agg1 = (
            dinv * jnp.dot(A, dinv * h1, preferred_element_type=jnp.float32)
            + (dinv * dinv) * h1
            + bc1_ref[...]
        )

        mean = jnp.sum(agg1, axis=0, keepdims=True) / N
        var = jnp.sum((agg1 - mean) ** 2, axis=0, keepdims=True) / N
        o1 = (gamma_ref[...] * (agg1 - mean) * jax.lax.rsqrt(var + 1e-5)
              + beta_ref[...])
        o1 = jnp.maximum(o1, 0.0)

        h2 = jnp.dot(o1, wc2_ref[...], preferred_element_type=jnp.float32)
        out_ref[...] = (
            dinv * jnp.dot(A, dinv * h2, preferred_element_type=jnp.float32)
            + (dinv * dinv) * h2
            + bc2_ref[...]
        )


def kernel(probs, bbox_coords, query_embeddings, node_embeddings,
           W1, b1, W2, b2, Wc1, bc1, Wc2, bc2, gamma, beta):
    f32 = jnp.float32
    W1c = W1[N + P:, :]

    const = lambda i: (0, 0)
    out = pl.pallas_call(
        _body,
        grid=(NBLK,),
        in_specs=[
            pl.BlockSpec((RB, N), lambda i: (i, 0)),
            pl.BlockSpec((RB, P), lambda i: (i, 0)),
            pl.BlockSpec((RB, 4), lambda i: (i, 0)),
            pl.BlockSpec((N + P + 4, MLPH), const),
            pl.BlockSpec((4, MLPH), const),
            pl.BlockSpec((1, MLPH), const),
            pl.BlockSpec((MLPH, N), const),
            pl.BlockSpec((1, N), const),
            pl.BlockSpec((N, N), const),
            pl.BlockSpec((N, HID), const),
            pl.BlockSpec((1, HID), const),
            pl.BlockSpec((HID, OUT), const),
            pl.BlockSpec((1, OUT), const),
            pl.BlockSpec((1, HID), const),
            pl.BlockSpec((1, HID), const),
        ],
        out_specs=pl.BlockSpec((N, OUT), const),
        out_shape=jax.ShapeDtypeStruct((N, OUT), f32),
        scratch_shapes=[pltpu.VMEM((N, N), f32)],
        compiler_params=pltpu.CompilerParams(
            dimension_semantics=("arbitrary",),
        ),
    )(query_embeddings, probs, bbox_coords, W1, W1c,
      b1.reshape(1, MLPH), W2, b2.reshape(1, N),
      node_embeddings, Wc1, bc1.reshape(1, HID), Wc2, bc2.reshape(1, OUT),
      gamma.reshape(1, HID), beta.reshape(1, HID))
    return out


# speedup vs baseline: 1.7059x; 1.0199x over previous
import jax
import jax.numpy as jnp
from jax.experimental import pallas as pl
from jax.experimental.pallas import tpu as pltpu

N = 1000
K = 32
RB = 200
NBLK = N // RB
P = 1001
MLPH = 512
HID = 128
OUT = 128


def _cumsum_lanes(x):
    rows, n = x.shape
    s = 1
    while s < n:
        shifted = jnp.concatenate(
            [jnp.zeros((rows, s), x.dtype), x[:, : n - s]], axis=1
        )
        x = x + shifted
        s *= 2
    return x


def _count_ge(key, c, ones8):
    rows, n = key.shape
    m = jnp.where(key >= c, 1.0, 0.0)
    t = m[:, 0:128]
    for j in range(1, 7):
        t = t + m[:, j * 128:(j + 1) * 128]
    tail = jnp.concatenate(
        [m[:, 896:n], jnp.zeros((rows, 1024 - n), jnp.float32)], axis=1
    )
    t = t + tail
    return jnp.dot(t, ones8, preferred_element_type=jnp.float32)[:, 0:1]


def _topk_search(adj):
    rows, n = adj.shape
    b = jax.lax.bitcast_convert_type(adj, jnp.int32)
    key = jnp.where(b >= 0, b, b ^ jnp.int32(0x7FFFFFFF))
    ones8 = jnp.ones((128, 8), jnp.float32)
    kf = jnp.float32(K)

    def body(i, carry):
        p, cnt = carry
        c = p + (jnp.int32(1) << (jnp.int32(31) - i))
        cc = jnp.sum((key >= c).astype(jnp.float32), axis=1, keepdims=True)
        ok = cc >= kf
        return jnp.where(ok, c, p), jnp.where(ok, cc, cnt)

    p0 = jnp.full((rows, 1), jnp.int32(-2147483648))
    cnt0 = jnp.full((rows, 1), jnp.float32(n))
    p, cnt = jax.lax.fori_loop(0, 32, body, (p0, cnt0), unroll=16)
    return key, p, cnt, kf


def _body(q_ref, p_ref, bb_ref, w1_ref, w1c_ref, b1_ref, w2_ref, b2_ref,
          ne_ref, wc1_ref, bc1_ref, wc2_ref, bc2_ref, gamma_ref, beta_ref,
          out_ref, a_scr):
    i = pl.program_id(0)

    h = jnp.dot(q_ref[...], w1_ref[0:N, :], preferred_element_type=jnp.float32)
    h = h + jnp.dot(p_ref[...], w1_ref[N:N + P, :],
                    preferred_element_type=jnp.float32)
    h = h + jnp.dot(bb_ref[...], w1c_ref[...],
                    preferred_element_type=jnp.float32)
    h = jnp.maximum(h + b1_ref[...], 0.0)
    adj = jnp.dot(h, w2_ref[...], preferred_element_type=jnp.float32) + b2_ref[...]

    key, p, cnt, kf = _topk_search(adj)
    rowsl = pl.ds(i * RB, RB)
    a_scr[rowsl, :] = jnp.where(key >= p, adj, 0.0)

    any_tie = jnp.sum(jnp.where(cnt != kf, 1.0, 0.0)) > 0.0

    @pl.when(any_tie)
    def _exact_ties():
        gt = key > p
        eq = key == p
        cgt = jnp.sum(gt.astype(jnp.int32), axis=1, keepdims=True)
        need = K - cgt
        eqcs = _cumsum_lanes(eq.astype(jnp.int32))
        mask = gt | (eq & ((cnt == kf) | (eqcs <= need)))
        a_scr[rowsl, :] = jnp.where(mask, adj, 0.0)

    @pl.when(i == NBLK - 1)
    def _gcn():
        A = a_scr[...]
        deg = 1.0 + jnp.sum(A, axis=1, keepdims=True)
        dinv = jnp.where(deg > 0, jax.lax.rsqrt(deg), 0.0)

        h1 = jnp.dot(ne_ref[...], wc1_ref[...],
                     preferred_element_type=jnp.float32)
        agg1 = (
            dinv * jnp.dot(A, dinv * h1, preferred_element_type=jnp.float32)
            + (dinv * dinv) * h1
            + bc1_ref[...]
        )

        mean = jnp.sum(agg1, axis=0, keepdims=True) / N
        var = jnp.sum((agg1 - mean) ** 2, axis=0, keepdims=True) / N
        o1 = (gamma_ref[...] * (agg1 - mean) * jax.lax.rsqrt(var + 1e-5)
              + beta_ref[...])
        o1 = jnp.maximum(o1, 0.0)

        h2 = jnp.dot(o1, wc2_ref[...], preferred_element_type=jnp.float32)
        out_ref[...] = (
            dinv * jnp.dot(A, dinv * h2, preferred_element_type=jnp.float32)
            + (dinv * dinv) * h2
            + bc2_ref[...]
        )


def kernel(probs, bbox_coords, query_embeddings, node_embeddings,
           W1, b1, W2, b2, Wc1, bc1, Wc2, bc2, gamma, beta):
    f32 = jnp.float32
    W1c = W1[N + P:, :]

    const = lambda i: (0, 0)
    out = pl.pallas_call(
        _body,
        grid=(NBLK,),
        in_specs=[
            pl.BlockSpec((RB, N), lambda i: (i, 0)),
            pl.BlockSpec((RB, P), lambda i: (i, 0)),
            pl.BlockSpec((RB, 4), lambda i: (i, 0)),
            pl.BlockSpec((N + P + 4, MLPH), const),
            pl.BlockSpec((4, MLPH), const),
            pl.BlockSpec((1, MLPH), const),
            pl.BlockSpec((MLPH, N), const),
            pl.BlockSpec((1, N), const),
            pl.BlockSpec((N, N), const),
            pl.BlockSpec((N, HID), const),
            pl.BlockSpec((1, HID), const),
            pl.BlockSpec((HID, OUT), const),
            pl.BlockSpec((1, OUT), const),
            pl.BlockSpec((1, HID), const),
            pl.BlockSpec((1, HID), const),
        ],
        out_specs=pl.BlockSpec((N, OUT), const),
        out_shape=jax.ShapeDtypeStruct((N, OUT), f32),
        scratch_shapes=[pltpu.VMEM((N, N), f32)],
        compiler_params=pltpu.CompilerParams(
            dimension_semantics=("arbitrary",),
        ),
    )(query_embeddings, probs, bbox_coords, W1, W1c,
      b1.reshape(1, MLPH), W2, b2.reshape(1, N),
      node_embeddings, Wc1, bc1.reshape(1, HID), Wc2, bc2.reshape(1, OUT),
      gamma.reshape(1, HID), beta.reshape(1, HID))
    return out


# speedup vs baseline: 1.7615x; 1.0326x over previous
import jax
import jax.numpy as jnp
from jax.experimental import pallas as pl
from jax.experimental.pallas import tpu as pltpu

N = 1000
K = 32
RB = 200
NBLK = N // RB
P = 1001
MLPH = 512
HID = 128
OUT = 128


def _cumsum_lanes(x):
    rows, n = x.shape
    s = 1
    while s < n:
        shifted = jnp.concatenate(
            [jnp.zeros((rows, s), x.dtype), x[:, : n - s]], axis=1
        )
        x = x + shifted
        s *= 2
    return x


def _count_ge(key, c, ones8):
    rows, n = key.shape
    m = jnp.where(key >= c, 1.0, 0.0)
    t = m[:, 0:128]
    for j in range(1, 7):
        t = t + m[:, j * 128:(j + 1) * 128]
    tail = jnp.concatenate(
        [m[:, 896:n], jnp.zeros((rows, 1024 - n), jnp.float32)], axis=1
    )
    t = t + tail
    return jnp.dot(t, ones8, preferred_element_type=jnp.float32)[:, 0:1]


def _topk_search(adj):
    rows, n = adj.shape
    b = jax.lax.bitcast_convert_type(adj, jnp.int32)
    key = jnp.where(b >= 0, b, b ^ jnp.int32(0x7FFFFFFF))
    ones8 = jnp.ones((128, 8), jnp.float32)
    kf = jnp.float32(K)

    def body(i, carry):
        p, cnt = carry
        c = p + (jnp.int32(1) << (jnp.int32(31) - i))
        cc = jnp.sum((key >= c).astype(jnp.float32), axis=1, keepdims=True)
        ok = cc >= kf
        return jnp.where(ok, c, p), jnp.where(ok, cc, cnt)

    p0 = jnp.full((rows, 1), jnp.int32(-2147483648))
    cnt0 = jnp.full((rows, 1), jnp.float32(n))
    p, cnt = jax.lax.fori_loop(0, 32, body, (p0, cnt0), unroll=32)
    return key, p, cnt, kf


def _body(q_ref, p_ref, bb_ref, w1_ref, w1c_ref, b1_ref, w2_ref, b2_ref,
          ne_ref, wc1_ref, bc1_ref, wc2_ref, bc2_ref, gamma_ref, beta_ref,
          out_ref, a_scr):
    i = pl.program_id(0)

    h = jnp.dot(q_ref[...], w1_ref[0:N, :], preferred_element_type=jnp.float32)
    h = h + jnp.dot(p_ref[...], w1_ref[N:N + P, :],
                    preferred_element_type=jnp.float32)
    h = h + jnp.dot(bb_ref[...], w1c_ref[...],
                    preferred_element_type=jnp.float32)
    h = jnp.maximum(h + b1_ref[...], 0.0)
    adj = jnp.dot(h, w2_ref[...], preferred_element_type=jnp.float32) + b2_ref[...]

    key, p, cnt, kf = _topk_search(adj)
    rowsl = pl.ds(i * RB, RB)
    a_scr[rowsl, :] = jnp.where(key >= p, adj, 0.0)

    any_tie = jnp.sum(jnp.where(cnt != kf, 1.0, 0.0)) > 0.0

    @pl.when(any_tie)
    def _exact_ties():
        gt = key > p
        eq = key == p
        cgt = jnp.sum(gt.astype(jnp.int32), axis=1, keepdims=True)
        need = K - cgt
        eqcs = _cumsum_lanes(eq.astype(jnp.int32))
        mask = gt | (eq & ((cnt == kf) | (eqcs <= need)))
        a_scr[rowsl, :] = jnp.where(mask, adj, 0.0)

    @pl.when(i == NBLK - 1)
    def _gcn():
        A = a_scr[...]
        deg = 1.0 + jnp.sum(A, axis=1, keepdims=True)
        dinv = jnp.where(deg > 0, jax.lax.rsqrt(deg), 0.0)

        h1 = jnp.dot(ne_ref[...], wc1_ref[...],
                     preferred_element_type=jnp.float32)
        agg1 = (
            dinv * jnp.dot(A, dinv * h1, preferred_element_type=jnp.float32)
            + (dinv * dinv) * h1
            + bc1_ref[...]
        )

        mean = jnp.sum(agg1, axis=0, keepdims=True) / N
        var = jnp.sum((agg1 - mean) ** 2, axis=0, keepdims=True) / N
        o1 = (gamma_ref[...] * (agg1 - mean) * jax.lax.rsqrt(var + 1e-5)
              + beta_ref[...])
        o1 = jnp.maximum(o1, 0.0)

        h2 = jnp.dot(o1, wc2_ref[...], preferred_element_type=jnp.float32)
        out_ref[...] = (
            dinv * jnp.dot(A, dinv * h2, preferred_element_type=jnp.float32)
            + (dinv * dinv) * h2
            + bc2_ref[...]
        )


def kernel(probs, bbox_coords, query_embeddings, node_embeddings,
           W1, b1, W2, b2, Wc1, bc1, Wc2, bc2, gamma, beta):
    f32 = jnp.float32
    W1c = W1[N + P:, :]

    const = lambda i: (0, 0)
    out = pl.pallas_call(
        _body,
        grid=(NBLK,),
        in_specs=[
            pl.BlockSpec((RB, N), lambda i: (i, 0)),
            pl.BlockSpec((RB, P), lambda i: (i, 0)),
            pl.BlockSpec((RB, 4), lambda i: (i, 0)),
            pl.BlockSpec((N + P + 4, MLPH), const),
            pl.BlockSpec((4, MLPH), const),
            pl.BlockSpec((1, MLPH), const),
            pl.BlockSpec((MLPH, N), const),
            pl.BlockSpec((1, N), const),
            pl.BlockSpec((N, N), const),
            pl.BlockSpec((N, HID), const),
            pl.BlockSpec((1, HID), const),
            pl.BlockSpec((HID, OUT), const),
            pl.BlockSpec((1, OUT), const),
            pl.BlockSpec((1, HID), const),
            pl.BlockSpec((1, HID), const),
        ],
        out_specs=pl.BlockSpec((N, OUT), const),
        out_shape=jax.ShapeDtypeStruct((N, OUT), f32),
        scratch_shapes=[pltpu.VMEM((N, N), f32)],
        compiler_params=pltpu.CompilerParams(
            dimension_semantics=("arbitrary",),
        ),
    )(query_embeddings, probs, bbox_coords, W1, W1c,
      b1.reshape(1, MLPH), W2, b2.reshape(1, N),
      node_embeddings, Wc1, bc1.reshape(1, HID), Wc2, bc2.reshape(1, OUT),
      gamma.reshape(1, HID), beta.reshape(1, HID))
    return out
